# baseline (speedup 1.0000x reference)
import functools
import math

import jax
import jax.numpy as jnp
from jax import lax
from jax.experimental import pallas as pl
from jax.experimental.pallas import tpu as pltpu
from jax.experimental.pallas import tpu_sc as plsc

D_MODEL = 64
VOCAB_SIZE = 1_000_000
SCALE = math.sqrt(D_MODEL)

NUM_CORES = 2
NUM_SUBCORES = 16
LANES = 16
NW = NUM_CORES * NUM_SUBCORES

B_TOTAL = 4096 * 200
B_PER_W = B_TOTAL // NW
CHUNK = 512
N_CHUNKS = B_PER_W // CHUNK
N_PAIRS = N_CHUNKS // 2


def _scale_rows(rows_ref):
    @pl.loop(0, CHUNK, unroll=4)
    def _(r):
        for j in range(D_MODEL // LANES):
            sl = pl.ds(j * LANES, LANES)
            rows_ref[r, sl] = rows_ref[r, sl] * SCALE


def _emb_body(table_hbm, idx_hbm, out_hbm,
              idx0, idx1, rows0, rows1, sem0, sem1):
    wid = lax.axis_index("s") * NUM_CORES + lax.axis_index("c")
    base = wid * B_PER_W

    def fetch_and_gather(chunk_id, idx_v, rows_v, sem):
        start = base + chunk_id * CHUNK
        pltpu.sync_copy(idx_hbm.at[pl.ds(start, CHUNK)], idx_v)
        return pltpu.async_copy(table_hbm.at[idx_v], rows_v, sem)

    def finish(chunk_id, idx_v, rows_v, sem):
        pltpu.make_async_copy(table_hbm.at[idx_v], rows_v, sem).wait()
        _scale_rows(rows_v)
        start = base + chunk_id * CHUNK
        pltpu.sync_copy(rows_v, out_hbm.at[pl.ds(start, CHUNK)])

    fetch_and_gather(0, idx0, rows0, sem0)

    @pl.loop(0, N_PAIRS)
    def _(p):
        g0 = p * 2
        fetch_and_gather(g0 + 1, idx1, rows1, sem1)
        finish(g0, idx0, rows0, sem0)

        @pl.when(p < N_PAIRS - 1)
        def _():
            fetch_and_gather(g0 + 2, idx0, rows0, sem0)

        finish(g0 + 1, idx1, rows1, sem1)


@functools.partial(jax.jit, static_argnames=())
def _embedding_lookup(x_flat, table):
    mesh = plsc.VectorSubcoreMesh(
        core_axis_name="c", subcore_axis_name="s",
        num_cores=NUM_CORES, num_subcores=NUM_SUBCORES)
    f = pl.kernel(
        _emb_body,
        out_type=jax.ShapeDtypeStruct((B_TOTAL, D_MODEL), jnp.float32),
        mesh=mesh,
        scratch_types=[
            pltpu.VMEM((CHUNK,), jnp.int32),
            pltpu.VMEM((CHUNK,), jnp.int32),
            pltpu.VMEM((CHUNK, D_MODEL), jnp.float32),
            pltpu.VMEM((CHUNK, D_MODEL), jnp.float32),
            pltpu.SemaphoreType.DMA,
            pltpu.SemaphoreType.DMA,
        ],
        compiler_params=pltpu.CompilerParams(use_tc_tiling_on_sc=False),
    )
    return f(table, x_flat)


def kernel(x, table):
    x_flat = x.reshape(-1).astype(jnp.int32)
    out = _embedding_lookup(x_flat, table)
    return out.reshape(x.shape + (D_MODEL,))


# speedup vs baseline: 1.0009x; 1.0009x over previous
import functools
import math

import jax
import jax.numpy as jnp
from jax import lax
from jax.experimental import pallas as pl
from jax.experimental.pallas import tpu as pltpu
from jax.experimental.pallas import tpu_sc as plsc

D_MODEL = 64
VOCAB_SIZE = 1_000_000
SCALE = math.sqrt(D_MODEL)

NUM_CORES = 2
NUM_SUBCORES = 16
LANES = 16
NW = NUM_CORES * NUM_SUBCORES

B_TOTAL = 4096 * 200
B_PER_W = B_TOTAL // NW
CHUNK = 512
N_CHUNKS = B_PER_W // CHUNK
N_PAIRS = N_CHUNKS // 2


def _scale_rows(rows_ref):
    @pl.loop(0, CHUNK, unroll=4)
    def _(r):
        for j in range(D_MODEL // LANES):
            sl = pl.ds(j * LANES, LANES)
            rows_ref[r, sl] = rows_ref[r, sl] * SCALE


def _emb_body(table_hbm, idx_hbm, out_hbm,
              idx0, idx1, rows0, rows1, sem0, sem1):
    wid = lax.axis_index("s") * NUM_CORES + lax.axis_index("c")
    base = wid * B_PER_W

    def fetch_and_gather(chunk_id, idx_v, rows_v, sem):
        start = base + chunk_id * CHUNK
        pltpu.sync_copy(idx_hbm.at[pl.ds(start, CHUNK)], idx_v)
        return pltpu.async_copy(table_hbm.at[idx_v], rows_v, sem)

    def finish(chunk_id, idx_v, rows_v, sem):
        pltpu.make_async_copy(table_hbm.at[idx_v], rows_v, sem).wait()
        _scale_rows(rows_v)
        start = base + chunk_id * CHUNK
        pltpu.sync_copy(rows_v, out_hbm.at[pl.ds(start, CHUNK)])

    fetch_and_gather(0, idx0, rows0, sem0)

    @pl.loop(0, N_PAIRS)
    def _(p):
        g0 = p * 2
        fetch_and_gather(g0 + 1, idx1, rows1, sem1)
        finish(g0, idx0, rows0, sem0)

        @pl.when(p < N_PAIRS - 1)
        def _():
            fetch_and_gather(g0 + 2, idx0, rows0, sem0)

        finish(g0 + 1, idx1, rows1, sem1)


@functools.partial(jax.jit, static_argnames=())
def _embedding_lookup(x_flat, table):
    mesh = plsc.VectorSubcoreMesh(
        core_axis_name="c", subcore_axis_name="s",
        num_cores=NUM_CORES, num_subcores=NUM_SUBCORES)
    f = pl.kernel(
        _emb_body,
        out_type=jax.ShapeDtypeStruct((B_TOTAL, D_MODEL), jnp.float32),
        mesh=mesh,
        scratch_types=[
            pltpu.VMEM((CHUNK,), jnp.int32),
            pltpu.VMEM((CHUNK,), jnp.int32),
            pltpu.VMEM((CHUNK, D_MODEL), jnp.float32),
            pltpu.VMEM((CHUNK, D_MODEL), jnp.float32),
            pltpu.SemaphoreType.DMA,
            pltpu.SemaphoreType.DMA,
        ],
        compiler_params=pltpu.CompilerParams(
            use_tc_tiling_on_sc=False, skip_device_barrier=True),
    )
    return f(table, x_flat)


def kernel(x, table):
    x_flat = x.reshape(-1).astype(jnp.int32)
    out = _embedding_lookup(x_flat, table)
    return out.reshape(x.shape + (D_MODEL,))
